# initial kernel scaffold (unmeasured)
import jax
import jax.numpy as jnp
from jax import lax
from jax.experimental import pallas as pl
from jax.experimental.pallas import tpu as pltpu

N_DEV = 8
B_PER = 512
D = 256
H_PER = 512

BF16 = jnp.bfloat16
F32 = jnp.float32


def kernel(x, Win0, Wout0, Win1, Wout1, Win2, Wout2):
    def body(x_ref, win0_ref, wout0_ref, win1_ref, wout1_ref, win2_ref,
             wout2_ref, out_ref, xfull, acc, rs_buf, ag_ss, ag_rs, rs_ss,
             rs_rs):
        me = lax.axis_index("i")
        right = lax.rem(me + 1, N_DEV)

        def ag_phase():
            for h in range(N_DEV - 1):
                so = lax.rem(me - h + N_DEV, N_DEV)
                ro = lax.rem(me - h - 1 + N_DEV, N_DEV)
                send_d = pltpu.make_async_remote_copy(
                    src_ref=xfull.at[so],
                    dst_ref=xfull.at[so],
                    send_sem=ag_ss.at[h],
                    recv_sem=ag_rs.at[h],
                    device_id=(right,),
                    device_id_type=pl.DeviceIdType.MESH,
                )
                send_d.start()
                recv_d = pltpu.make_async_remote_copy(
                    src_ref=xfull.at[ro],
                    dst_ref=xfull.at[ro],
                    send_sem=ag_ss.at[h],
                    recv_sem=ag_rs.at[h],
                    device_id=(right,),
                    device_id_type=pl.DeviceIdType.MESH,
                )
                recv_d.wait_recv()
                send_d.wait_send()

        def rs_phase():
            for s in range(N_DEV - 1):
                cs = lax.rem(me - s - 1 + N_DEV, N_DEV)
                cr = lax.rem(me - s - 2 + 2 * N_DEV, N_DEV)
                rdma = pltpu.make_async_remote_copy(
                    src_ref=acc.at[cs],
                    dst_ref=rs_buf.at[s],
                    send_sem=rs_ss.at[s],
                    recv_sem=rs_rs.at[s],
                    device_id=(right,),
                    device_id_type=pl.DeviceIdType.MESH,
                )
                rdma.start()
                rdma.wait()
                acc[cr] = acc[cr] + rs_buf[s]

        def compute_layer(win_ref, wout_ref):
            win = win_ref[:].astype(BF16)
            wout = wout_ref[:].astype(BF16)
            for c in range(N_DEV):
                h = jnp.dot(xfull[c], win, preferred_element_type=F32)
                h = jnp.maximum(h, 0.0).astype(BF16)
                acc[c] = jnp.dot(h, wout, preferred_element_type=F32)

        xfull[me] = x_ref[:].astype(BF16)
        ag_phase()

        for win_ref, wout_ref in ((win0_ref, wout0_ref),
                                  (win1_ref, wout1_ref),
                                  (win2_ref, wout2_ref)):
            compute_layer(win_ref, wout_ref)
            rs_phase()
            xfull[me] = acc[me].astype(BF16)
            ag_phase()

        for c in range(N_DEV):
            out_ref[pl.ds(c * B_PER, B_PER), :] = xfull[c].astype(F32)

    return pl.pallas_call(
        body,
        out_shape=jax.ShapeDtypeStruct((N_DEV * B_PER, D), F32),
        in_specs=[pl.BlockSpec(memory_space=pltpu.VMEM)] * 7,
        out_specs=pl.BlockSpec(memory_space=pltpu.VMEM),
        scratch_shapes=[
            pltpu.VMEM((N_DEV, B_PER, D), BF16),
            pltpu.VMEM((N_DEV, B_PER, D), F32),
            pltpu.VMEM((N_DEV - 1, B_PER, D), F32),
            pltpu.SemaphoreType.DMA((N_DEV - 1,)),
            pltpu.SemaphoreType.DMA((N_DEV - 1,)),
            pltpu.SemaphoreType.DMA((N_DEV - 1,)),
            pltpu.SemaphoreType.DMA((N_DEV - 1,)),
        ],
        compiler_params=pltpu.CompilerParams(collective_id=0),
    )(x, Win0, Wout0, Win1, Wout1, Win2, Wout2)


# baseline (device time: 310914 ns/iter reference)
import jax
import jax.numpy as jnp
from jax import lax
from jax.experimental import pallas as pl
from jax.experimental.pallas import tpu as pltpu

N_DEV = 8
B_PER = 512
D = 256
H_PER = 512

BF16 = jnp.bfloat16
F32 = jnp.float32


def kernel(x, Win0, Wout0, Win1, Wout1, Win2, Wout2):
    def body(x_ref, win0_ref, wout0_ref, win1_ref, wout1_ref, win2_ref,
             wout2_ref, out_ref, xfull, acc, rs_buf, ag_ss, ag_rs, rs_ss,
             rs_rs):
        me = lax.axis_index("i")
        right = lax.rem(me + 1, N_DEV)

        def ag_phase():
            for h in range(N_DEV - 1):
                so = lax.rem(me - h + N_DEV, N_DEV)
                ro = lax.rem(me - h - 1 + N_DEV, N_DEV)
                send_d = pltpu.make_async_remote_copy(
                    src_ref=xfull.at[so],
                    dst_ref=xfull.at[so],
                    send_sem=ag_ss.at[h],
                    recv_sem=ag_rs.at[h],
                    device_id=(right,),
                    device_id_type=pl.DeviceIdType.MESH,
                )
                send_d.start()
                recv_d = pltpu.make_async_remote_copy(
                    src_ref=xfull.at[ro],
                    dst_ref=xfull.at[ro],
                    send_sem=ag_ss.at[h],
                    recv_sem=ag_rs.at[h],
                    device_id=(right,),
                    device_id_type=pl.DeviceIdType.MESH,
                )
                recv_d.wait_recv()
                send_d.wait_send()

        def rs_phase():
            for s in range(N_DEV - 1):
                cs = lax.rem(me - s - 1 + N_DEV, N_DEV)
                cr = lax.rem(me - s - 2 + 2 * N_DEV, N_DEV)
                rdma = pltpu.make_async_remote_copy(
                    src_ref=acc.at[cs],
                    dst_ref=rs_buf.at[s],
                    send_sem=rs_ss.at[s],
                    recv_sem=rs_rs.at[s],
                    device_id=(right,),
                    device_id_type=pl.DeviceIdType.MESH,
                )
                rdma.start()
                rdma.wait()
                acc[cr] = acc[cr] + rs_buf[s]

        def compute_layer(win_ref, wout_ref):
            win = win_ref[:].astype(BF16)
            wout = wout_ref[:].astype(BF16)
            for c in range(N_DEV):
                h = jnp.dot(xfull[c], win, preferred_element_type=F32)
                h = jnp.maximum(h, 0.0).astype(BF16)
                acc[c] = jnp.dot(h, wout, preferred_element_type=F32)

        xfull[me] = x_ref[:].astype(BF16)
        ag_phase()

        for win_ref, wout_ref in ((win0_ref, wout0_ref),
                                  (win1_ref, wout1_ref),
                                  (win2_ref, wout2_ref)):
            compute_layer(win_ref, wout_ref)
            rs_phase()
            xfull[me] = acc[me].astype(BF16)
            ag_phase()

        for c in range(N_DEV):
            out_ref[pl.ds(c * B_PER, B_PER), :] = xfull[c].astype(F32)

    return pl.pallas_call(
        body,
        out_shape=jax.ShapeDtypeStruct((N_DEV * B_PER, D), F32),
        in_specs=[pl.BlockSpec(memory_space=pltpu.VMEM)] * 7,
        out_specs=pl.BlockSpec(memory_space=pltpu.VMEM),
        scratch_shapes=[
            pltpu.VMEM((N_DEV, B_PER, D), BF16),
            pltpu.VMEM((N_DEV, B_PER, D), F32),
            pltpu.VMEM((N_DEV - 1, B_PER, D), F32),
            pltpu.SemaphoreType.DMA((N_DEV - 1,)),
            pltpu.SemaphoreType.DMA((N_DEV - 1,)),
            pltpu.SemaphoreType.DMA((N_DEV - 1,)),
            pltpu.SemaphoreType.DMA((N_DEV - 1,)),
        ],
    )(x, Win0, Wout0, Win1, Wout1, Win2, Wout2)


# device time: 197861 ns/iter; 1.5714x vs baseline; 1.5714x over previous
import jax
import jax.numpy as jnp
from jax import lax
from jax.experimental import pallas as pl
from jax.experimental.pallas import tpu as pltpu

N_DEV = 8
B_PER = 512
D = 256
H_PER = 512

BF16 = jnp.bfloat16
F32 = jnp.float32

AG_MASKS = (1, 3, 4)
RS_MASKS = (4, 3, 1)
RS_SLOT = (0, 4, 6)


def kernel(x, Win0, Wout0, Win1, Wout1, Win2, Wout2):
    def body(x_ref, win0_ref, wout0_ref, win1_ref, wout1_ref, win2_ref,
             wout2_ref, out_ref, xfull, acc, rs_buf, snd, ag_ss, ag_rs,
             rs_ss, rs_rs):
        me = lax.axis_index("i")

        def rows(chunk_start, n):
            return pl.ds(chunk_start * B_PER, n * B_PER)

        def ag_step(s):
            bs = 1 << s
            myb = jnp.bitwise_and(me, N_DEV - bs)
            pb = jnp.bitwise_xor(myb, bs)
            partner = (jnp.bitwise_xor(me, AG_MASKS[s]),)
            send_d = pltpu.make_async_remote_copy(
                src_ref=xfull.at[rows(myb, bs)],
                dst_ref=xfull.at[rows(myb, bs)],
                send_sem=ag_ss.at[s],
                recv_sem=ag_rs.at[s],
                device_id=partner,
                device_id_type=pl.DeviceIdType.MESH,
            )
            send_d.start()
            recv_d = pltpu.make_async_remote_copy(
                src_ref=xfull.at[rows(pb, bs)],
                dst_ref=xfull.at[rows(pb, bs)],
                send_sem=ag_ss.at[s],
                recv_sem=ag_rs.at[s],
                device_id=partner,
                device_id_type=pl.DeviceIdType.MESH,
            )
            return send_d, recv_d

        def rs_phase():
            for s in range(3):
                bs = 4 >> s
                myb = jnp.bitwise_and(me, N_DEV - bs)
                pb = jnp.bitwise_xor(myb, bs)
                slot = rows(RS_SLOT[s], bs)
                snd[slot] = acc[rows(pb, bs)].astype(BF16)
                rdma = pltpu.make_async_remote_copy(
                    src_ref=snd.at[slot],
                    dst_ref=rs_buf.at[slot],
                    send_sem=rs_ss.at[s],
                    recv_sem=rs_rs.at[s],
                    device_id=(jnp.bitwise_xor(me, RS_MASKS[s]),),
                    device_id_type=pl.DeviceIdType.MESH,
                )
                rdma.start()
                rdma.wait()
                mine = rows(myb, bs)
                acc[mine] = acc[mine] + rs_buf[slot].astype(F32)

        def compute_block(chunk_start, n, win, wout):
            r = rows(chunk_start, n)
            h = jnp.dot(xfull[r], win, preferred_element_type=F32)
            h = jnp.maximum(h, 0.0).astype(BF16)
            acc[r] = jnp.dot(h, wout, preferred_element_type=F32)

        xfull[rows(me, 1)] = x_ref[:].astype(BF16)

        for win_ref, wout_ref in ((win0_ref, wout0_ref),
                                  (win1_ref, wout1_ref),
                                  (win2_ref, wout2_ref)):
            win = win_ref[:].astype(BF16)
            wout = wout_ref[:].astype(BF16)
            s0, r0 = ag_step(0)
            compute_block(me, 1, win, wout)
            r0.wait_recv()
            s0.wait_send()
            s1, r1 = ag_step(1)
            compute_block(jnp.bitwise_xor(me, 1), 1, win, wout)
            r1.wait_recv()
            s1.wait_send()
            s2, r2 = ag_step(2)
            b1 = jnp.bitwise_xor(jnp.bitwise_and(me, N_DEV - 2), 2)
            compute_block(b1, 2, win, wout)
            r2.wait_recv()
            s2.wait_send()
            b2 = jnp.bitwise_xor(jnp.bitwise_and(me, N_DEV - 4), 4)
            compute_block(b2, 4, win, wout)
            rs_phase()
            xfull[rows(me, 1)] = acc[rows(me, 1)].astype(BF16)

        for s in range(3):
            sd, rd = ag_step(s)
            rd.wait_recv()
            sd.wait_send()
        out_ref[:] = xfull[:].astype(F32)

    return pl.pallas_call(
        body,
        out_shape=jax.ShapeDtypeStruct((N_DEV * B_PER, D), F32),
        in_specs=[pl.BlockSpec(memory_space=pltpu.VMEM)] * 7,
        out_specs=pl.BlockSpec(memory_space=pltpu.VMEM),
        scratch_shapes=[
            pltpu.VMEM((N_DEV * B_PER, D), BF16),
            pltpu.VMEM((N_DEV * B_PER, D), F32),
            pltpu.VMEM((7 * B_PER, D), BF16),
            pltpu.VMEM((7 * B_PER, D), BF16),
            pltpu.SemaphoreType.DMA((3,)),
            pltpu.SemaphoreType.DMA((3,)),
            pltpu.SemaphoreType.DMA((3,)),
            pltpu.SemaphoreType.DMA((3,)),
        ],
    )(x, Win0, Wout0, Win1, Wout1, Win2, Wout2)


# device time: 128283 ns/iter; 2.4237x vs baseline; 1.5424x over previous
import jax
import jax.numpy as jnp
from jax import lax
from jax.experimental import pallas as pl
from jax.experimental.pallas import tpu as pltpu

N_DEV = 8
R = 256
D = 256
BF16 = jnp.bfloat16
F32 = jnp.float32

A_AG = (1, 3, 4)
B_AG = (3, 4, 1)
A_RS = (4, 3, 1)
B_RS = (1, 4, 3)
RS_SLOT = (0, 4, 6)
SIGMA = (0, 4, 5, 1, 2, 6, 7, 3)


def kernel(x, Win0, Wout0, Win1, Wout1, Win2, Wout2):
    def body(x_ref, win0_ref, wout0_ref, win1_ref, wout1_ref, win2_ref,
             wout2_ref, out_ref, xa, xb, acca, accb, rba, rbb, snda, sndb,
             ag_ss_a, ag_rs_a, ag_ss_b, ag_rs_b,
             rs_ss_a, rs_rs_a, rs_ss_b, rs_rs_b):
        me = lax.axis_index("i")
        b0 = jnp.bitwise_and(me, 1)
        b1 = jnp.bitwise_and(lax.shift_right_logical(me, 1), 1)
        b2 = jnp.bitwise_and(lax.shift_right_logical(me, 2), 1)
        sb = b1 + 2 * b2 + 4 * jnp.bitwise_xor(b0, b1)

        def rows(slot_start, n):
            return pl.ds(slot_start * R, n * R)

        def ag_step(s, xbuf, own, masks, ss, rs):
            bs = 1 << s
            myb = jnp.bitwise_and(own, N_DEV - bs)
            pb = jnp.bitwise_xor(myb, bs)
            partner = (jnp.bitwise_xor(me, masks[s]),)
            send_d = pltpu.make_async_remote_copy(
                src_ref=xbuf.at[rows(myb, bs)],
                dst_ref=xbuf.at[rows(myb, bs)],
                send_sem=ss.at[s],
                recv_sem=rs.at[s],
                device_id=partner,
                device_id_type=pl.DeviceIdType.MESH,
            )
            send_d.start()
            recv_d = pltpu.make_async_remote_copy(
                src_ref=xbuf.at[rows(pb, bs)],
                dst_ref=xbuf.at[rows(pb, bs)],
                send_sem=ss.at[s],
                recv_sem=rs.at[s],
                device_id=partner,
                device_id_type=pl.DeviceIdType.MESH,
            )
            return send_d, recv_d

        def rs_step_start(s, accbuf, sndbuf, rbuf, own, masks, ss, rs):
            bs = 4 >> s
            myb = jnp.bitwise_and(own, N_DEV - bs)
            pb = jnp.bitwise_xor(myb, bs)
            slot = rows(RS_SLOT[s], bs)
            sndbuf[slot] = accbuf[rows(pb, bs)].astype(BF16)
            rdma = pltpu.make_async_remote_copy(
                src_ref=sndbuf.at[slot],
                dst_ref=rbuf.at[slot],
                send_sem=ss.at[s],
                recv_sem=rs.at[s],
                device_id=(jnp.bitwise_xor(me, masks[s]),),
                device_id_type=pl.DeviceIdType.MESH,
            )
            rdma.start()
            return rdma, myb

        def rs_step_finish(s, rdma, accbuf, rbuf, myb):
            bs = 4 >> s
            rdma.wait()
            slot = rows(RS_SLOT[s], bs)
            mine = rows(myb, bs)
            accbuf[mine] = accbuf[mine] + rbuf[slot].astype(F32)

        def compute_block(xbuf, accbuf, slot_start, n, win, wout):
            r = rows(slot_start, n)
            h = jnp.dot(xbuf[r], win, preferred_element_type=F32)
            h = jnp.maximum(h, 0.0).astype(BF16)
            accbuf[r] = jnp.dot(h, wout, preferred_element_type=F32)

        xa[rows(me, 1)] = x_ref[pl.ds(0, R), :].astype(BF16)
        xb[rows(sb, 1)] = x_ref[pl.ds(R, R), :].astype(BF16)

        for win_ref, wout_ref in ((win0_ref, wout0_ref),
                                  (win1_ref, wout1_ref),
                                  (win2_ref, wout2_ref)):
            win = win_ref[:].astype(BF16)
            wout = wout_ref[:].astype(BF16)

            sa0, ra0 = ag_step(0, xa, me, A_AG, ag_ss_a, ag_rs_a)
            sb0, rb0 = ag_step(0, xb, sb, B_AG, ag_ss_b, ag_rs_b)
            compute_block(xa, acca, me, 1, win, wout)
            compute_block(xb, accb, sb, 1, win, wout)
            ra0.wait_recv()
            rb0.wait_recv()
            sa0.wait_send()
            sb0.wait_send()

            sa1, ra1 = ag_step(1, xa, me, A_AG, ag_ss_a, ag_rs_a)
            sb1, rb1 = ag_step(1, xb, sb, B_AG, ag_ss_b, ag_rs_b)
            compute_block(xa, acca, jnp.bitwise_xor(me, 1), 1, win, wout)
            compute_block(xb, accb, jnp.bitwise_xor(sb, 1), 1, win, wout)
            ra1.wait_recv()
            rb1.wait_recv()
            sa1.wait_send()
            sb1.wait_send()

            sa2, ra2 = ag_step(2, xa, me, A_AG, ag_ss_a, ag_rs_a)
            sb2, rb2 = ag_step(2, xb, sb, B_AG, ag_ss_b, ag_rs_b)
            a1b = jnp.bitwise_xor(jnp.bitwise_and(me, N_DEV - 2), 2)
            b1b = jnp.bitwise_xor(jnp.bitwise_and(sb, N_DEV - 2), 2)
            compute_block(xa, acca, a1b, 2, win, wout)
            compute_block(xb, accb, b1b, 2, win, wout)
            ra2.wait_recv()
            rb2.wait_recv()
            sa2.wait_send()
            sb2.wait_send()

            a2b = jnp.bitwise_xor(jnp.bitwise_and(me, N_DEV - 4), 4)
            b2b = jnp.bitwise_xor(jnp.bitwise_and(sb, N_DEV - 4), 4)
            compute_block(xa, acca, a2b, 4, win, wout)
            compute_block(xb, accb, b2b, 4, win, wout)

            for s in range(3):
                da, mba = rs_step_start(s, acca, snda, rba, me, A_RS,
                                        rs_ss_a, rs_rs_a)
                db, mbb = rs_step_start(s, accb, sndb, rbb, sb, B_RS,
                                        rs_ss_b, rs_rs_b)
                rs_step_finish(s, da, acca, rba, mba)
                rs_step_finish(s, db, accb, rbb, mbb)

            xa[rows(me, 1)] = acca[rows(me, 1)].astype(BF16)
            xb[rows(sb, 1)] = accb[rows(sb, 1)].astype(BF16)

        for s in range(3):
            sa, ra = ag_step(s, xa, me, A_AG, ag_ss_a, ag_rs_a)
            sbd, rbd = ag_step(s, xb, sb, B_AG, ag_ss_b, ag_rs_b)
            ra.wait_recv()
            rbd.wait_recv()
            sa.wait_send()
            sbd.wait_send()

        for c in range(N_DEV):
            out_ref[pl.ds(c * 2 * R, R), :] = xa[rows(c, 1)].astype(F32)
            out_ref[pl.ds(c * 2 * R + R, R), :] = (
                xb[rows(SIGMA[c], 1)].astype(F32))

    sem3 = pltpu.SemaphoreType.DMA((3,))
    return pl.pallas_call(
        body,
        out_shape=jax.ShapeDtypeStruct((N_DEV * 2 * R, D), F32),
        in_specs=[pl.BlockSpec(memory_space=pltpu.VMEM)] * 7,
        out_specs=pl.BlockSpec(memory_space=pltpu.VMEM),
        scratch_shapes=[
            pltpu.VMEM((N_DEV * R, D), BF16),
            pltpu.VMEM((N_DEV * R, D), BF16),
            pltpu.VMEM((N_DEV * R, D), F32),
            pltpu.VMEM((N_DEV * R, D), F32),
            pltpu.VMEM((7 * R, D), BF16),
            pltpu.VMEM((7 * R, D), BF16),
            pltpu.VMEM((7 * R, D), BF16),
            pltpu.VMEM((7 * R, D), BF16),
            sem3, sem3, sem3, sem3,
            sem3, sem3, sem3, sem3,
        ],
    )(x, Win0, Wout0, Win1, Wout1, Win2, Wout2)


# device time: 84582 ns/iter; 3.6759x vs baseline; 1.5167x over previous
import jax
import jax.numpy as jnp
from jax import lax
from jax.experimental import pallas as pl
from jax.experimental.pallas import tpu as pltpu

N_DEV = 8
B_PER = 512
D = 256
H_BLK = 512
R = 256

BF16 = jnp.bfloat16
F32 = jnp.float32

ORDERS = ((1, 3, 4), (3, 4, 1), (4, 1, 3))
A_AG = (1, 3, 4)
B_AG = (3, 4, 1)
SIGMA1 = (0, 4, 5, 1, 2, 6, 7, 3)


def kernel(x, Win0, Wout0, Win1, Wout1, Win2, Wout2):
    def body(x_ref, win0_ref, wout0_ref, win1_ref, wout1_ref, win2_ref,
             wout2_ref, out_ref, winf0, winf1, winf2, woutf0, woutf1,
             woutf2, xa, xb,
             gw_ss, gw_rs, go_ss, go_rs, ag_ss_a, ag_rs_a, ag_ss_b, ag_rs_b):
        me = lax.axis_index("i")
        b0 = jnp.bitwise_and(me, 1)
        b1 = jnp.bitwise_and(lax.shift_right_logical(me, 1), 1)
        b2 = jnp.bitwise_and(lax.shift_right_logical(me, 2), 1)
        b01 = jnp.bitwise_xor(b0, b1)
        sl = (me,
              b1 + 2 * b2 + 4 * b01,
              b2 + 2 * b01 + 4 * b1)

        winfs = (winf0, winf1, winf2)
        woutfs = (woutf0, woutf1, woutf2)

        def gather_step(l, s):
            bs = 1 << s
            myb = jnp.bitwise_and(sl[l], N_DEV - bs)
            pb = jnp.bitwise_xor(myb, bs)
            partner = (jnp.bitwise_xor(me, ORDERS[l][s]),)

            def descs(blk):
                cols = pl.ds(blk * H_BLK, bs * H_BLK)
                w_d = pltpu.make_async_remote_copy(
                    src_ref=winfs[l].at[cols],
                    dst_ref=winfs[l].at[cols],
                    send_sem=gw_ss.at[l, s],
                    recv_sem=gw_rs.at[l, s],
                    device_id=partner,
                    device_id_type=pl.DeviceIdType.MESH,
                )
                o_d = pltpu.make_async_remote_copy(
                    src_ref=woutfs[l].at[cols],
                    dst_ref=woutfs[l].at[cols],
                    send_sem=go_ss.at[l, s],
                    recv_sem=go_rs.at[l, s],
                    device_id=partner,
                    device_id_type=pl.DeviceIdType.MESH,
                )
                return w_d, o_d

            ws, os_ = descs(myb)
            ws.start()
            os_.start()
            wr, orr = descs(pb)
            return ws, os_, wr, orr

        def wait_step(ds4):
            ws, os_, wr, orr = ds4
            wr.wait_recv()
            orr.wait_recv()
            ws.wait_send()
            os_.wait_send()

        xloc = x_ref[:].astype(BF16)
        for l, (wi, wo) in enumerate(((win0_ref, wout0_ref),
                                      (win1_ref, wout1_ref),
                                      (win2_ref, wout2_ref))):
            cols = pl.ds(sl[l] * H_BLK, H_BLK)
            winfs[l][cols] = wi[:]
            woutfs[l][cols] = wo[:].astype(BF16)

        def hdot(xv, wslice):
            return lax.dot_general(
                xv, wslice, (((1,), (1,)), ((), ())),
                preferred_element_type=F32)

        def l0_block(acc, blk, nb):
            cols = pl.ds(blk * H_BLK, nb * H_BLK)
            h = hdot(xloc, winf0[cols])
            h = jnp.maximum(h, 0.0).astype(BF16)
            c = jnp.dot(h, woutf0[cols], preferred_element_type=F32)
            return c if acc is None else acc + c

        st0 = [gather_step(l, 0) for l in range(3)]
        acc = l0_block(None, sl[0], 1)
        for d4 in st0:
            wait_step(d4)
        st1 = [gather_step(l, 1) for l in range(3)]
        acc = l0_block(acc, jnp.bitwise_xor(sl[0], 1), 1)
        for d4 in st1:
            wait_step(d4)
        st2 = [gather_step(l, 2) for l in range(3)]
        blk1 = jnp.bitwise_xor(jnp.bitwise_and(sl[0], N_DEV - 2), 2)
        acc = l0_block(acc, blk1, 2)
        for d4 in st2:
            wait_step(d4)
        blk2 = jnp.bitwise_xor(jnp.bitwise_and(sl[0], N_DEV - 4), 4)
        acc = l0_block(acc, blk2, 4)

        for l in (1, 2):
            xcur = acc.astype(BF16)
            h = hdot(xcur, winfs[l][:])
            h = jnp.maximum(h, 0.0).astype(BF16)
            acc = jnp.dot(h, woutfs[l][:], preferred_element_type=F32)

        sbf = sl[1]
        xa[pl.ds(me * R, R)] = acc[:R].astype(BF16)
        xb[pl.ds(sbf * R, R)] = acc[R:].astype(BF16)

        def ag_step(s, xbuf, own, masks, ss, rs):
            bs = 1 << s
            myb = jnp.bitwise_and(own, N_DEV - bs)
            pb = jnp.bitwise_xor(myb, bs)
            partner = (jnp.bitwise_xor(me, masks[s]),)
            send_d = pltpu.make_async_remote_copy(
                src_ref=xbuf.at[pl.ds(myb * R, bs * R)],
                dst_ref=xbuf.at[pl.ds(myb * R, bs * R)],
                send_sem=ss.at[s],
                recv_sem=rs.at[s],
                device_id=partner,
                device_id_type=pl.DeviceIdType.MESH,
            )
            send_d.start()
            recv_d = pltpu.make_async_remote_copy(
                src_ref=xbuf.at[pl.ds(pb * R, bs * R)],
                dst_ref=xbuf.at[pl.ds(pb * R, bs * R)],
                send_sem=ss.at[s],
                recv_sem=rs.at[s],
                device_id=partner,
                device_id_type=pl.DeviceIdType.MESH,
            )
            return send_d, recv_d

        for s in range(3):
            sa, ra = ag_step(s, xa, me, A_AG, ag_ss_a, ag_rs_a)
            sbd, rbd = ag_step(s, xb, sbf, B_AG, ag_ss_b, ag_rs_b)
            ra.wait_recv()
            rbd.wait_recv()
            sa.wait_send()
            sbd.wait_send()

        for c in range(N_DEV):
            out_ref[pl.ds(c * 2 * R, R), :] = (
                xa[pl.ds(c * R, R)].astype(F32))
            out_ref[pl.ds(c * 2 * R + R, R), :] = (
                xb[pl.ds(SIGMA1[c] * R, R)].astype(F32))

    sem3 = pltpu.SemaphoreType.DMA((3,))
    sem33 = pltpu.SemaphoreType.DMA((3, 3))
    return pl.pallas_call(
        body,
        out_shape=jax.ShapeDtypeStruct((N_DEV * B_PER, D), F32),
        in_specs=[pl.BlockSpec(memory_space=pltpu.VMEM)] * 7,
        out_specs=pl.BlockSpec(memory_space=pltpu.VMEM),
        scratch_shapes=[
            pltpu.VMEM((N_DEV * H_BLK, D), BF16),
            pltpu.VMEM((N_DEV * H_BLK, D), BF16),
            pltpu.VMEM((N_DEV * H_BLK, D), BF16),
            pltpu.VMEM((N_DEV * H_BLK, D), BF16),
            pltpu.VMEM((N_DEV * H_BLK, D), BF16),
            pltpu.VMEM((N_DEV * H_BLK, D), BF16),
            pltpu.VMEM((N_DEV * R, D), BF16),
            pltpu.VMEM((N_DEV * R, D), BF16),
            sem33, sem33,
            sem33, sem33,
            sem3, sem3, sem3, sem3,
        ],
    )(x, Win0.T.astype(BF16), Wout0, Win1.T.astype(BF16), Wout1,
      Win2.T.astype(BF16), Wout2)
